# baseline (speedup 1.0000x reference)
import functools

import jax
import jax.numpy as jnp
from jax import lax
from jax.experimental import pallas as pl
from jax.experimental.pallas import tpu as pltpu
from jax.experimental.pallas import tpu_sc as plsc

VOCAB_SIZE = 1000000
D = 64
HD = D // 2
B = 16384
CHUNK = 128


def _make_gather():
    info = plsc.get_sparse_core_info()
    nw = info.num_cores * info.num_subcores
    b_per_w = B // nw
    n_chunks = b_per_w // CHUNK
    mesh = plsc.VectorSubcoreMesh(core_axis_name="c", subcore_axis_name="s")

    @functools.partial(
        pl.kernel,
        mesh=mesh,
        out_type=jax.ShapeDtypeStruct((B, D), jnp.float32),
        scratch_types=[
            pltpu.VMEM((b_per_w,), jnp.int32),
            pltpu.VMEM((b_per_w, HD), jnp.float32),
            pltpu.VMEM((b_per_w, HD), jnp.float32),
            pltpu.VMEM((b_per_w, D), jnp.float32),
            pltpu.SemaphoreType.DMA,
        ],
        compiler_params=pltpu.CompilerParams(use_tc_tiling_on_sc=False),
    )
    def k(tab_l, tab_r, idx_hbm, out_hbm, idx_v, rl_v, rr_v, out_v, sem):
        wid = lax.axis_index("s") * info.num_cores + lax.axis_index("c")
        base = wid * b_per_w
        pltpu.sync_copy(idx_hbm.at[pl.ds(base, b_per_w)], idx_v)

        for c in range(n_chunks):
            sl = pl.ds(c * CHUNK, CHUNK)
            pltpu.async_copy(tab_l.at[idx_v.at[sl]], rl_v.at[sl], sem)
            pltpu.async_copy(tab_r.at[idx_v.at[sl]], rr_v.at[sl], sem)
        pltpu.make_async_copy(tab_l.at[pl.ds(0, b_per_w)], rl_v, sem).wait()
        pltpu.make_async_copy(tab_r.at[pl.ds(0, b_per_w)], rr_v, sem).wait()

        def weave(g, _):
            gbase = g * 16
            for j in range(16):
                i = gbase + j
                for col in range(HD // 16):
                    sl = pl.ds(col * 16, 16)
                    out_v.at[i][pl.ds(col * 16, 16)] = rl_v.at[i][sl]
                    out_v.at[i][pl.ds(HD + col * 16, 16)] = rr_v.at[i][sl]
            return 0

        lax.fori_loop(0, b_per_w // 16, weave, 0)
        pltpu.sync_copy(out_v, out_hbm.at[pl.ds(base, b_per_w)])

    return k


_gather = _make_gather()


def kernel(x, table):
    xi = x.astype(jnp.int32)
    return _gather(table[:, :HD], table[:, HD:], xi)


# speedup vs baseline: 3.8739x; 3.8739x over previous
import functools

import jax
import jax.numpy as jnp
from jax import lax
from jax.experimental import pallas as pl
from jax.experimental.pallas import tpu as pltpu
from jax.experimental.pallas import tpu_sc as plsc

VOCAB_SIZE = 1000000
D = 64
B = 16384


def _make_gather():
    info = plsc.get_sparse_core_info()
    nw = info.num_cores * info.num_subcores
    b_per_w = B // nw
    mesh = plsc.VectorSubcoreMesh(core_axis_name="c", subcore_axis_name="s")

    @functools.partial(
        pl.kernel,
        mesh=mesh,
        out_type=jax.ShapeDtypeStruct((B, D), jnp.float32),
        scratch_types=[
            pltpu.VMEM((b_per_w,), jnp.int32),
            pltpu.VMEM((b_per_w, D), jnp.float32),
            pltpu.SemaphoreType.DMA,
        ],
    )
    def k(table_hbm, idx_hbm, out_hbm, idx_v, rows_v, sem):
        wid = lax.axis_index("s") * info.num_cores + lax.axis_index("c")
        base = wid * b_per_w
        pltpu.sync_copy(idx_hbm.at[pl.ds(base, b_per_w)], idx_v)

        def body(g, _):
            base_i = g * 16
            v = idx_v[pl.ds(base_i, 16)]
            for j in range(16):
                row = v[j]
                pltpu.async_copy(
                    table_hbm.at[pl.ds(row, 1), :],
                    rows_v.at[pl.ds(base_i + j, 1), :],
                    sem,
                )
            return 0

        lax.fori_loop(0, b_per_w // 16, body, 0)
        pltpu.make_async_copy(
            table_hbm.at[pl.ds(0, b_per_w), :], rows_v, sem
        ).wait()
        pltpu.sync_copy(rows_v, out_hbm.at[pl.ds(base, b_per_w)])

    return k


_gather = _make_gather()


def kernel(x, table):
    return _gather(table, x.astype(jnp.int32))
